# baseline (speedup 1.0000x reference)
import functools

import jax
import jax.numpy as jnp
from jax import lax
from jax.experimental import pallas as pl
from jax.experimental.pallas import tpu as pltpu
from jax.experimental.pallas import tpu_sc as plsc

_VOCAB = 1000
_HIDDEN = 64
_B = 1024
_L = 50
_TOK = _B * _L
_HP = 128
_NC, _NS = 2, 16
_NW = _NC * _NS
_TPW = _TOK // _NW
_CH = 80
_NCH = _TPW // _CH
_NBUF = 2


def _embed_gather(tab_p, idx2):
    tpw = idx2.shape[1]
    nch = tpw // _CH
    mesh = plsc.VectorSubcoreMesh(
        core_axis_name="c", subcore_axis_name="s",
        num_cores=_NC, num_subcores=_NS)

    @functools.partial(
        pl.kernel,
        out_type=jax.ShapeDtypeStruct((_NW * tpw, _HP), jnp.float32),
        mesh=mesh,
        scratch_types=[
            pltpu.VMEM((tpw,), jnp.int32),
            pltpu.VMEM((_CH, _HP), jnp.float32),
            pltpu.VMEM((_CH, _HP), jnp.float32),
            pltpu.SemaphoreType.DMA,
            pltpu.SemaphoreType.DMA,
            pltpu.SemaphoreType.DMA,
            pltpu.SemaphoreType.DMA,
        ],
    )
    def k(tab_hbm, idx_hbm, out_hbm, idx_v, buf0, buf1, g0, g1, s0, s1):
        wid = lax.axis_index("s") * _NC + lax.axis_index("c")
        base = wid * tpw
        pltpu.sync_copy(idx_hbm.at[wid], idx_v)
        bufs = (buf0, buf1)
        gsems = (g0, g1)
        ssems = (s0, s1)

        def gather(c, b):
            return pltpu.make_async_copy(
                tab_hbm.at[idx_v.at[pl.ds(c * _CH, _CH)]], bufs[b], gsems[b])

        def scatter(c, b):
            return pltpu.make_async_copy(
                bufs[b], out_hbm.at[pl.ds(base + c * _CH, _CH)], ssems[b])

        gather(0, 0).start()
        gather(1, 1).start()

        def body(gi, carry):
            for b in range(_NBUF):
                c = _NBUF * gi + b
                gather(c, b).wait()
                scatter(c, b).start()
            for b in range(_NBUF):
                c = _NBUF * gi + b
                scatter(c, b).wait()

                @pl.when(gi + 1 < nch // _NBUF)
                def _():
                    gather(c + _NBUF, b).start()

            return carry

        lax.fori_loop(0, nch // _NBUF, body, 0)

    return k(tab_p, idx2)


def _head_body(emb_ref, w_ref, b_ref, out_ref):
    x = emb_ref[...][:, :_HIDDEN]
    y = lax.dot_general(
        w_ref[...], x, (((1,), (1,)), ((), ())),
        preferred_element_type=jnp.float32)
    out_ref[...] = (y + b_ref[...])[None]


def _head_body2(prev_ref, emb_ref, w_ref, b_ref, out_ref):
    del prev_ref
    _head_body(emb_ref, w_ref, b_ref, out_ref)


_LH = 10


def _head_matmul_lo(embeds_p, head_w, head_b2):
    return pl.pallas_call(
        _head_body,
        grid=(_LH,),
        in_specs=[
            pl.BlockSpec((_B, _HP), lambda l: (l, 0)),
            pl.BlockSpec((_VOCAB, _HIDDEN), lambda l: (0, 0)),
            pl.BlockSpec((_VOCAB, 1), lambda l: (0, 0)),
        ],
        out_specs=pl.BlockSpec((1, _VOCAB, _B), lambda l: (l, 0, 0)),
        out_shape=jax.ShapeDtypeStruct((_L, _VOCAB, _B), jnp.float32),
    )(embeds_p, head_w, head_b2)


def _head_matmul_hi(prev, embeds_p, head_w, head_b2):
    return pl.pallas_call(
        _head_body2,
        grid=(_L - _LH,),
        in_specs=[
            pl.BlockSpec(memory_space=pl.ANY),
            pl.BlockSpec((_B, _HP), lambda l: (l, 0)),
            pl.BlockSpec((_VOCAB, _HIDDEN), lambda l: (0, 0)),
            pl.BlockSpec((_VOCAB, 1), lambda l: (0, 0)),
        ],
        out_specs=pl.BlockSpec((1, _VOCAB, _B), lambda l: (l + _LH, 0, 0)),
        out_shape=jax.ShapeDtypeStruct((_L, _VOCAB, _B), jnp.float32),
        input_output_aliases={0: 0},
    )(prev, embeds_p, head_w, head_b2)


def kernel(input_ids, emb_table, head_w, head_b):
    ids_lmajor = input_ids.astype(jnp.int32).T.reshape(_TOK)
    tab_p = jnp.pad(emb_table, ((0, 0), (0, _HP - _HIDDEN)))
    half = _LH * _B
    emb_lo = _embed_gather(tab_p, ids_lmajor[:half].reshape(_NW, half // _NW))
    emb_hi = _embed_gather(
        tab_p, ids_lmajor[half:].reshape(_NW, (_TOK - half) // _NW))
    b2 = head_b.reshape(_VOCAB, 1)
    out_lo = _head_matmul_lo(emb_lo, head_w, b2)
    out_t = _head_matmul_hi(out_lo, emb_hi, head_w, b2)
    return jnp.transpose(out_t, (2, 0, 1))


# speedup vs baseline: 1.0245x; 1.0245x over previous
import functools

import jax
import jax.numpy as jnp
from jax import lax
from jax.experimental import pallas as pl
from jax.experimental.pallas import tpu as pltpu
from jax.experimental.pallas import tpu_sc as plsc

_VOCAB = 1000
_HIDDEN = 64
_B = 1024
_L = 50
_TOK = _B * _L
_HP = 128
_NC, _NS = 2, 16
_NW = _NC * _NS
_TPW = _TOK // _NW
_CH = 80
_NCH = _TPW // _CH
_NBUF = 2


def _embed_gather(tab_p, idx2):
    tpw = idx2.shape[1]
    nch = tpw // _CH
    mesh = plsc.VectorSubcoreMesh(
        core_axis_name="c", subcore_axis_name="s",
        num_cores=_NC, num_subcores=_NS)

    @functools.partial(
        pl.kernel,
        out_type=jax.ShapeDtypeStruct((_NW * tpw, _HP), jnp.float32),
        mesh=mesh,
        scratch_types=[
            pltpu.VMEM((tpw,), jnp.int32),
            pltpu.VMEM((_CH, _HP), jnp.float32),
            pltpu.VMEM((_CH, _HP), jnp.float32),
            pltpu.SemaphoreType.DMA,
            pltpu.SemaphoreType.DMA,
            pltpu.SemaphoreType.DMA,
            pltpu.SemaphoreType.DMA,
        ],
    )
    def k(tab_hbm, idx_hbm, out_hbm, idx_v, buf0, buf1, g0, g1, s0, s1):
        wid = lax.axis_index("s") * _NC + lax.axis_index("c")
        base = wid * tpw
        pltpu.sync_copy(idx_hbm.at[wid], idx_v)
        bufs = (buf0, buf1)
        gsems = (g0, g1)
        ssems = (s0, s1)

        def gather(c, b):
            return pltpu.make_async_copy(
                tab_hbm.at[idx_v.at[pl.ds(c * _CH, _CH)]], bufs[b], gsems[b])

        def scatter(c, b):
            return pltpu.make_async_copy(
                bufs[b], out_hbm.at[pl.ds(base + c * _CH, _CH)], ssems[b])

        gather(0, 0).start()
        gather(1, 1).start()

        def body(gi, carry):
            for b in range(_NBUF):
                c = _NBUF * gi + b
                gather(c, b).wait()
                scatter(c, b).start()
            for b in range(_NBUF):
                c = _NBUF * gi + b
                scatter(c, b).wait()

                @pl.when(gi + 1 < nch // _NBUF)
                def _():
                    gather(c + _NBUF, b).start()

            return carry

        lax.fori_loop(0, nch // _NBUF, body, 0)

    return k(tab_p, idx2)


def _head_body(emb_ref, w_ref, b_ref, out_ref):
    x = emb_ref[...][:, :_HIDDEN]
    y = lax.dot_general(
        w_ref[...], x, (((1,), (1,)), ((), ())),
        preferred_element_type=jnp.float32)
    out_ref[...] = (y + b_ref[...])[None]


def _head_body2(prev_ref, emb_ref, w_ref, b_ref, out_ref):
    del prev_ref
    _head_body(emb_ref, w_ref, b_ref, out_ref)


_LH = 20


def _head_matmul_lo(embeds_p, head_w, head_b2):
    return pl.pallas_call(
        _head_body,
        grid=(_LH,),
        in_specs=[
            pl.BlockSpec((_B, _HP), lambda l: (l, 0)),
            pl.BlockSpec((_VOCAB, _HIDDEN), lambda l: (0, 0)),
            pl.BlockSpec((_VOCAB, 1), lambda l: (0, 0)),
        ],
        out_specs=pl.BlockSpec((1, _VOCAB, _B), lambda l: (l, 0, 0)),
        out_shape=jax.ShapeDtypeStruct((_L, _VOCAB, _B), jnp.float32),
    )(embeds_p, head_w, head_b2)


def _head_matmul_hi(prev, embeds_p, head_w, head_b2):
    return pl.pallas_call(
        _head_body2,
        grid=(_L - _LH,),
        in_specs=[
            pl.BlockSpec(memory_space=pl.ANY),
            pl.BlockSpec((_B, _HP), lambda l: (l, 0)),
            pl.BlockSpec((_VOCAB, _HIDDEN), lambda l: (0, 0)),
            pl.BlockSpec((_VOCAB, 1), lambda l: (0, 0)),
        ],
        out_specs=pl.BlockSpec((1, _VOCAB, _B), lambda l: (l + _LH, 0, 0)),
        out_shape=jax.ShapeDtypeStruct((_L, _VOCAB, _B), jnp.float32),
        input_output_aliases={0: 0},
    )(prev, embeds_p, head_w, head_b2)


def kernel(input_ids, emb_table, head_w, head_b):
    ids_lmajor = input_ids.astype(jnp.int32).T.reshape(_TOK)
    tab_p = jnp.pad(emb_table, ((0, 0), (0, _HP - _HIDDEN)))
    half = _LH * _B
    emb_lo = _embed_gather(tab_p, ids_lmajor[:half].reshape(_NW, half // _NW))
    emb_hi = _embed_gather(
        tab_p, ids_lmajor[half:].reshape(_NW, (_TOK - half) // _NW))
    b2 = head_b.reshape(_VOCAB, 1)
    out_lo = _head_matmul_lo(emb_lo, head_w, b2)
    out_t = _head_matmul_hi(out_lo, emb_hi, head_w, b2)
    return jnp.transpose(out_t, (2, 0, 1))
